# Initial kernel scaffold; baseline (speedup 1.0000x reference)
import jax
import jax.numpy as jnp
from jax.experimental import pallas as pl


def kernel(x_num, x_cat, tables, W1, b1, W2, b2, W3, b3):
    raise NotImplementedError("write your pallas kernel here")


# speedup vs baseline: 8.0675x; 8.0675x over previous
import functools

import jax
import jax.numpy as jnp
from jax import lax
from jax.experimental import pallas as pl
from jax.experimental.pallas import tpu as pltpu
from jax.experimental.pallas import tpu_sc as plsc

_F = 26
_V = 100000
_E = 32
_NNUM = 13
_B = 16384

_NC = 2
_NS = 16
_NW = _NC * _NS
_R = _B * _F
_RPW = _R // _NW
_IPS = 128
_SPG = 8
_GROUP = _IPS * _SPG
_NG = _RPW // _GROUP


def _gather_body(table_hbm, idx_hbm, out_hbm, idx_v, rows_v, sem_g):
    wid = lax.axis_index("s") * _NC + lax.axis_index("c")
    base = pl.multiple_of(wid * _RPW, _GROUP)
    pltpu.sync_copy(idx_hbm.at[wid], idx_v)

    def group(g, carry):
        row0 = pl.multiple_of(base + g * _GROUP, _GROUP)
        copies = []
        for j in range(_SPG):
            copies.append(pltpu.async_copy(
                table_hbm.at[idx_v.at[g * _SPG + j]],
                rows_v.at[pl.ds(j * _IPS, _IPS)],
                sem_g))
        for c in copies:
            c.wait()
        pltpu.sync_copy(rows_v, out_hbm.at[pl.ds(row0, _GROUP)])
        return carry

    lax.fori_loop(0, _NG, group, 0)


def _sc_gather(table2d, idx3d):
    mesh = plsc.VectorSubcoreMesh(core_axis_name="c", subcore_axis_name="s")
    k = functools.partial(
        pl.kernel,
        out_type=jax.ShapeDtypeStruct((_R, _E), jnp.float32),
        mesh=mesh,
        scratch_types=[
            pltpu.VMEM((_RPW // _IPS, _IPS), jnp.int32),
            pltpu.VMEM((_GROUP, _E), jnp.float32),
            pltpu.SemaphoreType.DMA,
        ],
        compiler_params=pltpu.CompilerParams(use_tc_tiling_on_sc=False),
    )(_gather_body)
    return k(table2d, idx3d)


def _mlp_body(xn_ref, emb_ref, w1n_ref, w1e_ref, b1_ref, w2_ref, b2_ref,
              w3_ref, b3_ref, out_ref):
    h1 = jnp.dot(emb_ref[...], w1e_ref[...], preferred_element_type=jnp.float32)
    h1 = h1 + jnp.dot(xn_ref[...], w1n_ref[...], preferred_element_type=jnp.float32)
    h1 = jnp.maximum(h1 + b1_ref[...], 0.0)
    h2 = jnp.maximum(
        jnp.dot(h1, w2_ref[...], preferred_element_type=jnp.float32) + b2_ref[...],
        0.0)
    out_ref[...] = (
        jnp.dot(h2, w3_ref[...], preferred_element_type=jnp.float32) + b3_ref[...])


def _tc_mlp(x_num, emb_cat, w1n_t, w1e_t, b1, w2_t, b2, w3_t, b3):
    bb = 2048
    grid = (_B // bb,)
    d_e = _F * _E
    return pl.pallas_call(
        _mlp_body,
        grid=grid,
        in_specs=[
            pl.BlockSpec((bb, _NNUM), lambda i: (i, 0)),
            pl.BlockSpec((bb, d_e), lambda i: (i, 0)),
            pl.BlockSpec((_NNUM, 128), lambda i: (0, 0)),
            pl.BlockSpec((d_e, 128), lambda i: (0, 0)),
            pl.BlockSpec((1, 128), lambda i: (0, 0)),
            pl.BlockSpec((128, 64), lambda i: (0, 0)),
            pl.BlockSpec((1, 64), lambda i: (0, 0)),
            pl.BlockSpec((64, 1), lambda i: (0, 0)),
            pl.BlockSpec((1, 1), lambda i: (0, 0)),
        ],
        out_specs=pl.BlockSpec((bb, 1), lambda i: (i, 0)),
        out_shape=jax.ShapeDtypeStruct((_B, 1), jnp.float32),
    )(x_num, emb_cat, w1n_t, w1e_t, b1, w2_t, b2, w3_t, b3)


def kernel(x_num, x_cat, tables, W1, b1, W2, b2, W3, b3):
    flat_idx = (x_cat.astype(jnp.int32)
                + (jnp.arange(_F, dtype=jnp.int32) * _V)[None, :])
    idx3d = flat_idx.reshape(_NW, _RPW // _IPS, _IPS)
    table2d = tables.reshape(_F * _V, _E)
    emb_flat = _sc_gather(table2d, idx3d)
    emb_cat = emb_flat.reshape(_B, _F * _E)
    out = _tc_mlp(
        x_num, emb_cat,
        W1[:, :_NNUM].T, W1[:, _NNUM:].T, b1.reshape(1, -1),
        W2.T, b2.reshape(1, -1),
        W3.T, b3.reshape(1, -1),
    )
    return out
